# Initial kernel scaffold; baseline (speedup 1.0000x reference)
import functools

import jax
import jax.numpy as jnp
from jax import lax
from jax.experimental import pallas as pl
from jax.experimental.pallas import tpu as pltpu
from jax.experimental.pallas import tpu_sc as plsc

B = 8
L = 16384
IN = 64
OUT = 16
K = 4
FLAT = B * L

NC = 2
NS = 16
NW = NC * NS
RW = FLAT // NW
CH = 128
NCH = RW // CH
LANES = 16


def _mm_body(x_ref, w_ref, b_ref, o_ref):
    o_ref[:] = (
        jnp.dot(x_ref[:], w_ref[:], preferred_element_type=jnp.float32)
        + b_ref[0:1, :]
    )


def _project(x_flat, w_cat, bvec):
    blk = 2048
    grid = FLAT // blk
    return pl.pallas_call(
        _mm_body,
        grid=(grid,),
        in_specs=[
            pl.BlockSpec((blk, IN), lambda i: (i, 0)),
            pl.BlockSpec((IN, K * OUT), lambda i: (0, 0)),
            pl.BlockSpec((8, K * OUT), lambda i: (0, 0)),
        ],
        out_specs=pl.BlockSpec((blk, K * OUT), lambda i: (i, 0)),
        out_shape=jax.ShapeDtypeStruct((FLAT, K * OUT), jnp.float32),
    )(x_flat, w_cat, bvec)


def _sc_body(z_hbm, idxt_hbm, out_hbm, idx_v, gidx_v, buf_v, obuf_v, sem):
    wid = lax.axis_index("s") * NC + lax.axis_index("c")
    batch = wid // (L // RW)
    j0 = (wid % (L // RW)) * RW
    row0 = wid * RW

    pltpu.sync_copy(idxt_hbm.at[:, pl.ds(j0, RW)], idx_v)

    base = batch * (L * K)

    def idx_body(i, _):
        off = pl.multiple_of(i * LANES, LANES)
        for k in range(K):
            v = idx_v[k, pl.ds(off, LANES)]
            gidx_v[k, pl.ds(off, LANES)] = v * K + (base + k)
        return 0

    lax.fori_loop(0, RW // LANES, idx_body, 0)

    def chunk_body(c, _):
        coff = pl.multiple_of(c * CH, CH)
        copies = [
            pltpu.make_async_copy(
                z_hbm.at[gidx_v.at[k, pl.ds(coff, CH)]], buf_v.at[k], sem)
            for k in range(K)
        ]
        for cp in copies:
            cp.start()
        for cp in copies:
            cp.wait()

        def row_body(r, _):
            acc = buf_v[0, r, :] + buf_v[1, r, :]
            acc = acc + buf_v[2, r, :]
            acc = acc + buf_v[3, r, :]
            obuf_v[r, :] = jnp.maximum(acc, 0.0)
            return 0

        lax.fori_loop(0, CH, row_body, 0)
        pltpu.sync_copy(obuf_v, out_hbm.at[pl.ds(row0 + coff, CH)])
        return 0

    lax.fori_loop(0, NCH, chunk_body, 0)


@functools.partial(
    pl.kernel,
    out_type=jax.ShapeDtypeStruct((FLAT, OUT), jnp.float32),
    mesh=plsc.VectorSubcoreMesh(
        core_axis_name="c", subcore_axis_name="s", num_cores=NC,
        num_subcores=NS),
    scratch_types=[
        pltpu.VMEM((K, RW), jnp.int32),
        pltpu.VMEM((K, RW), jnp.int32),
        pltpu.VMEM((K, CH, OUT), jnp.float32),
        pltpu.VMEM((CH, OUT), jnp.float32),
        pltpu.SemaphoreType.DMA,
    ],
)
def _sc_gather_reduce(z_hbm, idxt_hbm, out_hbm, idx_v, gidx_v, buf_v,
                      obuf_v, sem):
    _sc_body(z_hbm, idxt_hbm, out_hbm, idx_v, gidx_v, buf_v, obuf_v, sem)


def kernel(x, mask, bias, index_tensor):
    x_flat = x.reshape(FLAT, IN)
    w_cat = jnp.transpose(mask, (1, 0, 2)).reshape(IN, K * OUT)
    brow = jnp.concatenate(
        [jnp.full((OUT,), bias[-1], jnp.float32),
         jnp.zeros((K * OUT - OUT,), jnp.float32)])
    bvec = jnp.broadcast_to(brow, (8, K * OUT))

    z = _project(x_flat, w_cat, bvec)
    z_rows = z.reshape(FLAT * K, OUT)
    idxt = jnp.transpose(index_tensor).astype(jnp.int32)

    out = _sc_gather_reduce(z_rows, idxt)
    return out.reshape(B, L, OUT)


# speedup vs baseline: 5.1408x; 5.1408x over previous
import functools

import jax
import jax.numpy as jnp
from jax import lax
from jax.experimental import pallas as pl
from jax.experimental.pallas import tpu as pltpu
from jax.experimental.pallas import tpu_sc as plsc

B = 8
L = 16384
IN = 64
OUT = 16
K = 4
FLAT = B * L

NC = 2
NS = 16
NW = NC * NS
RW = FLAT // NW
CH = 128
NCH = RW // CH
LANES = 16


def _mm_body(x_ref, w_ref, b_ref, o_ref):
    o_ref[:] = (
        jnp.dot(x_ref[:], w_ref[:], preferred_element_type=jnp.float32)
        + b_ref[0:1, :]
    )


def _project(x_flat, w_cat, bvec):
    blk = 2048
    grid = FLAT // blk
    return pl.pallas_call(
        _mm_body,
        grid=(grid,),
        in_specs=[
            pl.BlockSpec((blk, IN), lambda i: (i, 0)),
            pl.BlockSpec((IN, K * OUT), lambda i: (0, 0)),
            pl.BlockSpec((8, K * OUT), lambda i: (0, 0)),
        ],
        out_specs=pl.BlockSpec((blk, K * OUT), lambda i: (i, 0)),
        out_shape=jax.ShapeDtypeStruct((FLAT, K * OUT), jnp.float32),
    )(x_flat, w_cat, bvec)


def _sc_body(z_hbm, idxt_hbm, out_hbm, idx_v, gidx_v, buf_v, obuf_v, sem):
    wid = lax.axis_index("s") * NC + lax.axis_index("c")
    batch = wid // (L // RW)
    j0 = (wid % (L // RW)) * RW
    row0 = wid * RW

    pltpu.sync_copy(idxt_hbm.at[:, pl.ds(j0, RW)], idx_v)

    base = batch * (L * K)

    def idx_body(i, _):
        off = pl.multiple_of(i * LANES, LANES)
        for k in range(K):
            v = idx_v[k, pl.ds(off, LANES)]
            gidx_v[k, pl.ds(off, LANES)] = v * K + (base + k)
        return 0

    lax.fori_loop(0, RW // LANES, idx_body, 0)

    def chunk_body(c, _):
        coff = pl.multiple_of(c * CH, CH)
        copies = [
            pltpu.make_async_copy(
                z_hbm.at[gidx_v.at[k, pl.ds(coff, CH)]], buf_v.at[k], sem)
            for k in range(K)
        ]
        for cp in copies:
            cp.start()
        for cp in copies:
            cp.wait()

        def row_body(r, _):
            acc = buf_v[0, r, :] + buf_v[1, r, :]
            acc = acc + buf_v[2, r, :]
            acc = acc + buf_v[3, r, :]
            obuf_v[r, :] = jnp.maximum(acc, 0.0)
            return 0

        lax.fori_loop(0, CH, row_body, 0)
        pltpu.sync_copy(obuf_v, out_hbm.at[pl.ds(row0 + coff, CH)])
        return 0

    lax.fori_loop(0, NCH, chunk_body, 0)


@functools.cache
def _sc_gather_reduce():
    return pl.kernel(
        _sc_body,
        out_type=jax.ShapeDtypeStruct((FLAT, OUT), jnp.float32),
        mesh=plsc.VectorSubcoreMesh(
            core_axis_name="c", subcore_axis_name="s", num_cores=NC,
            num_subcores=NS),
        scratch_types=[
            pltpu.VMEM((K, RW), jnp.int32),
            pltpu.VMEM((K, RW), jnp.int32),
            pltpu.VMEM((K, CH, OUT), jnp.float32),
            pltpu.VMEM((CH, OUT), jnp.float32),
            pltpu.SemaphoreType.DMA,
        ],
        compiler_params=pltpu.CompilerParams(use_tc_tiling_on_sc=False),
    )


def kernel(x, mask, bias, index_tensor):
    x_flat = x.reshape(FLAT, IN)
    w_cat = jnp.transpose(mask, (1, 0, 2)).reshape(IN, K * OUT)
    brow = jnp.concatenate(
        [jnp.full((OUT,), bias[-1], jnp.float32),
         jnp.zeros((K * OUT - OUT,), jnp.float32)])
    bvec = jnp.broadcast_to(brow, (8, K * OUT))

    z = _project(x_flat, w_cat, bvec)
    z_rows = z.reshape(FLAT * K, OUT)
    idxt = jnp.transpose(index_tensor).astype(jnp.int32)

    out = _sc_gather_reduce()(z_rows, idxt)
    return out.reshape(B, L, OUT)
